# baseline (speedup 1.0000x reference)
import functools

import jax
import jax.numpy as jnp
from jax import lax
from jax.experimental import pallas as pl
from jax.experimental.pallas import tpu as pltpu
from jax.experimental.pallas import tpu_sc as plsc

N = 10000
E = 320000
IN_CH = 128
HID = 8
D = IN_CH + HID
HP = 16
NITER = 3
EPS = 1e-5

CHUNK = 128
KSUB = 4
MACRO = CHUNK * KSUB
NMACRO = E // MACRO
NCHUNKS = E // CHUNK
NWORKER = 32
NTILE = 16

_mesh = plsc.VectorSubcoreMesh(core_axis_name="c", subcore_axis_name="s")
_sc_params = pltpu.CompilerParams(use_tc_tiling_on_sc=False)
_sc_params_tc = pltpu.CompilerParams(use_tc_tiling_on_sc=True)


@functools.partial(
    pl.kernel,
    out_type=(jax.ShapeDtypeStruct((E, IN_CH), jnp.float32),
              jax.ShapeDtypeStruct((E, IN_CH), jnp.float32)),
    mesh=_mesh,
    scratch_types=[
        pltpu.VMEM((KSUB, CHUNK), jnp.int32),
        pltpu.VMEM((MACRO, IN_CH), jnp.float32),
        pltpu.SemaphoreType.DMA,
    ],
    compiler_params=_sc_params_tc,
)
def _sc_gather_x(x_hbm, s2_hbm, t2_hbm, xs_hbm, xe_hbm, idx_v, rows_v, sem):
    wid = lax.axis_index("s") * 2 + lax.axis_index("c")
    niter = (NMACRO + NWORKER - 1) // NWORKER

    def body(i, carry):
        m = wid + i * NWORKER

        @pl.when(m < NMACRO)
        def _():
            for idx_hbm, out_hbm in ((s2_hbm, xs_hbm), (t2_hbm, xe_hbm)):
                pltpu.sync_copy(idx_hbm.at[pl.ds(m * KSUB, KSUB)], idx_v)
                copies = [
                    pltpu.async_copy(x_hbm.at[idx_v.at[j]],
                                     rows_v.at[pl.ds(j * CHUNK, CHUNK)], sem)
                    for j in range(KSUB)
                ]
                for cp in copies:
                    cp.wait()
                pltpu.sync_copy(rows_v, out_hbm.at[pl.ds(m * MACRO, MACRO)])

        return carry

    lax.fori_loop(0, niter, body, 0)


@functools.partial(
    pl.kernel,
    out_type=(jax.ShapeDtypeStruct((E, HP), jnp.float32),
              jax.ShapeDtypeStruct((E, HP), jnp.float32)),
    mesh=_mesh,
    scratch_types=[
        pltpu.VMEM((KSUB, CHUNK), jnp.int32),
        pltpu.VMEM((MACRO, HP), jnp.float32),
        pltpu.SemaphoreType.DMA,
    ],
    compiler_params=_sc_params,
)
def _sc_gather_a8(a_hbm, s2_hbm, t2_hbm, as_hbm, ae_hbm, idx_v, rows_v, sem):
    wid = lax.axis_index("s") * 2 + lax.axis_index("c")
    niter = (NMACRO + NWORKER - 1) // NWORKER

    def body(i, carry):
        m = wid + i * NWORKER

        @pl.when(m < NMACRO)
        def _():
            for idx_hbm, out_hbm in ((s2_hbm, as_hbm), (t2_hbm, ae_hbm)):
                pltpu.sync_copy(idx_hbm.at[pl.ds(m * KSUB, KSUB)], idx_v)
                copies = [
                    pltpu.async_copy(a_hbm.at[idx_v.at[j]],
                                     rows_v.at[pl.ds(j * CHUNK, CHUNK)], sem)
                    for j in range(KSUB)
                ]
                for cp in copies:
                    cp.wait()
                pltpu.sync_copy(rows_v, out_hbm.at[pl.ds(m * MACRO, MACRO)])

        return carry

    lax.fori_loop(0, niter, body, 0)


_ROWS_PER_TILE = N // NTILE
_WCHUNK = 125


_SNITER = (NCHUNKS + NTILE - 1) // NTILE


@functools.partial(
    pl.kernel,
    out_type=(jax.ShapeDtypeStruct((N, IN_CH), jnp.float32),
              jax.ShapeDtypeStruct((N, HP), jnp.float32),
              jax.ShapeDtypeStruct((N, IN_CH), jnp.float32),
              jax.ShapeDtypeStruct((N, HP), jnp.float32)),
    mesh=_mesh,
    scratch_types=[
        pltpu.VMEM((2, CHUNK), jnp.int32),
        pltpu.VMEM((2, CHUNK, IN_CH), jnp.float32),
        pltpu.VMEM((2, CHUNK, HP), jnp.float32),
        pltpu.VMEM_SHARED((N, IN_CH), jnp.float32),
        pltpu.VMEM_SHARED((N, HP), jnp.float32),
        pltpu.SemaphoreType.DMA,
        pltpu.SemaphoreType.DMA,
    ],
    compiler_params=_sc_params,
)
def _sc_scatter(msw_hbm, msh_hbm, mtw_hbm, mth_hbm, s2_hbm, t2_hbm,
                zzw_hbm, zzh_hbm, miw_hbm, mih_hbm, mow_hbm, moh_hbm,
                idx_v, roww_v, rowh_v, accw_sh, acch_sh, sem0, sem1):
    cid = lax.axis_index("c")
    sid = lax.axis_index("s")
    tbase = sid * _ROWS_PER_TILE
    sems = (sem0, sem1)

    pltpu.sync_copy(zzw_hbm, roww_v.at[0])
    pltpu.sync_copy(zzh_hbm, rowh_v.at[0])
    for k in range(_ROWS_PER_TILE // _WCHUNK):
        pltpu.sync_copy(roww_v.at[0, pl.ds(0, _WCHUNK)],
                        accw_sh.at[pl.ds(tbase + k * _WCHUNK, _WCHUNK)])
        pltpu.sync_copy(rowh_v.at[0, pl.ds(0, _WCHUNK)],
                        acch_sh.at[pl.ds(tbase + k * _WCHUNK, _WCHUNK)])
    plsc.subcore_barrier()

    def _start(i, b):
        m = sid + i * NTILE

        @pl.when(m < NCHUNKS)
        def _():
            @pl.when(cid == 0)
            def _():
                pltpu.async_copy(t2_hbm.at[pl.ds(m, 1)],
                                 idx_v.at[pl.ds(b, 1)], sems[b])
                pltpu.async_copy(msw_hbm.at[pl.ds(m * CHUNK, CHUNK)],
                                 roww_v.at[b], sems[b])
                pltpu.async_copy(msh_hbm.at[pl.ds(m * CHUNK, CHUNK)],
                                 rowh_v.at[b], sems[b])

            @pl.when(cid == 1)
            def _():
                pltpu.async_copy(s2_hbm.at[pl.ds(m, 1)],
                                 idx_v.at[pl.ds(b, 1)], sems[b])
                pltpu.async_copy(mtw_hbm.at[pl.ds(m * CHUNK, CHUNK)],
                                 roww_v.at[b], sems[b])
                pltpu.async_copy(mth_hbm.at[pl.ds(m * CHUNK, CHUNK)],
                                 rowh_v.at[b], sems[b])

    def _wait(i, b):
        m = sid + i * NTILE

        @pl.when(m < NCHUNKS)
        def _():
            pltpu.make_async_copy(t2_hbm.at[pl.ds(0, 1)],
                                  idx_v.at[pl.ds(b, 1)], sems[b]).wait()
            pltpu.make_async_copy(msw_hbm.at[pl.ds(0, CHUNK)],
                                  roww_v.at[b], sems[b]).wait()
            pltpu.make_async_copy(msh_hbm.at[pl.ds(0, CHUNK)],
                                  rowh_v.at[b], sems[b]).wait()

    def _use(i, b):
        m = sid + i * NTILE

        @pl.when(m < NCHUNKS)
        def _():
            pltpu.sync_copy(roww_v.at[b], accw_sh.at[idx_v.at[b]], add=True)
            pltpu.sync_copy(rowh_v.at[b], acch_sh.at[idx_v.at[b]], add=True)

    _start(0, 0)
    _start(1, 1)

    def body(j, carry):
        for b in (0, 1):
            i = 2 * j + b
            _wait(i, b)
            _use(i, b)
            _start(i + 2, b)
        return carry

    lax.fori_loop(0, (_SNITER + 1) // 2, body, 0)
    plsc.subcore_barrier()

    for k in range(_ROWS_PER_TILE // _WCHUNK):
        pltpu.sync_copy(accw_sh.at[pl.ds(tbase + k * _WCHUNK, _WCHUNK)],
                        roww_v.at[0, pl.ds(0, _WCHUNK)])
        pltpu.sync_copy(acch_sh.at[pl.ds(tbase + k * _WCHUNK, _WCHUNK)],
                        rowh_v.at[0, pl.ds(0, _WCHUNK)])

        @pl.when(cid == 0)
        def _():
            pltpu.sync_copy(roww_v.at[0, pl.ds(0, _WCHUNK)],
                            miw_hbm.at[pl.ds(tbase + k * _WCHUNK, _WCHUNK)])
            pltpu.sync_copy(rowh_v.at[0, pl.ds(0, _WCHUNK)],
                            mih_hbm.at[pl.ds(tbase + k * _WCHUNK, _WCHUNK)])

        @pl.when(cid == 1)
        def _():
            pltpu.sync_copy(roww_v.at[0, pl.ds(0, _WCHUNK)],
                            mow_hbm.at[pl.ds(tbase + k * _WCHUNK, _WCHUNK)])
            pltpu.sync_copy(rowh_v.at[0, pl.ds(0, _WCHUNK)],
                            moh_hbm.at[pl.ds(tbase + k * _WCHUNK, _WCHUNK)])


BN = 1000
BE = 2000
EGRID = E // BE


def _ln(z, g, b):
    m = jnp.mean(z, axis=-1, keepdims=True)
    v = jnp.var(z, axis=-1, keepdims=True)
    return (z - m) * lax.rsqrt(v + EPS) * g + b


def _dot(a, b):
    return jnp.dot(a, b, preferred_element_type=jnp.float32)


def _inp_body(x_ref, w_ref, b_ref, g_ref, be_ref, o_ref):
    xb = x_ref[...]
    z = _dot(xb, w_ref[...]) + b_ref[...]
    z = jnp.tanh(_ln(z, g_ref[...], be_ref[...]))
    o_ref[...] = jnp.concatenate(
        [z, jnp.zeros((xb.shape[0], HP - HID), jnp.float32)], axis=1)


def _gx_body(xs_ref, xe_ref, wxs_ref, wxt_ref, o_ref):
    o_ref[...] = (_dot(xs_ref[...], wxs_ref[...])
                  + _dot(xe_ref[...], wxt_ref[...]))


def _edge_mlp(a8s, a8t, gx, ck, w1s8, w1t8, w2, w3, w4, b1, b2, b3, b4,
              g1, g2, g3, e1, e2, e3):
    z = (_dot(a8s[:, :HID], w1s8) + _dot(a8t[:, :HID], w1t8)
         + ck * gx + b1)
    z = jnp.tanh(_ln(z, g1, e1))
    z = _dot(z, w2) + b2
    z = jnp.tanh(_ln(z, g2, e2))
    z = _dot(z, w3) + b3
    z = jnp.tanh(_ln(z, g3, e3))
    return _dot(z, w4) + b4


def _make_edge_loop_body(ck):
    def body(a8s_ref, a8t_ref, gx_ref, xs_ref, xe_ref,
             w1s8, w1t8, w2, w3, w4, b1, b2, b3, b4, g1, g2, g3, e1, e2, e3,
             msw_ref, msh_ref, mtw_ref, mth_ref):
        a8s = a8s_ref[...]
        a8t = a8t_ref[...]
        logit = _edge_mlp(a8s, a8t, gx_ref[...], ck,
                          w1s8[...], w1t8[...], w2[...], w3[...], w4[...],
                          b1[...], b2[...], b3[...], b4[...], g1[...], g2[...],
                          g3[...], e1[...], e2[...], e3[...])
        e = jax.nn.sigmoid(logit)
        msw_ref[...] = (e * ck) * xs_ref[...]
        msh_ref[...] = e * a8s
        mtw_ref[...] = (e * ck) * xe_ref[...]
        mth_ref[...] = e * a8t

    return body


def _edge_final_body(a8s_ref, a8t_ref, gx_ref,
                     w1s8, w1t8, w2, w3, w4, b1, b2, b3, b4,
                     g1, g2, g3, e1, e2, e3, o_ref):
    ck = float(NITER + 1)
    o_ref[...] = _edge_mlp(a8s_ref[...], a8t_ref[...], gx_ref[...], ck,
                           w1s8[...], w1t8[...], w2[...], w3[...], w4[...],
                           b1[...], b2[...], b3[...], b4[...], g1[...],
                           g2[...], g3[...], e1[...], e2[...], e3[...])


def _make_node_body(ck):
    def body(miw_ref, mih_ref, mow_ref, moh_ref, a8_ref, x_ref,
             w1a8, w1aw, w1b8, w1bw, w1c8, w1cw, w2, w3, w4,
             b1, b2, b3, b4, g1, g2, g3, e1, e2, e3, o_ref):
        a8 = a8_ref[...]
        z = (_dot(mih_ref[...][:, :HID], w1a8[...])
             + _dot(miw_ref[...], w1aw[...])
             + _dot(moh_ref[...][:, :HID], w1b8[...])
             + _dot(mow_ref[...], w1bw[...])
             + _dot(a8[:, :HID], w1c8[...])
             + ck * _dot(x_ref[...], w1cw[...])
             + b1[...])
        z = jnp.tanh(_ln(z, g1[...], e1[...]))
        z = _dot(z, w2[...]) + b2[...]
        z = jnp.tanh(_ln(z, g2[...], e2[...]))
        z = _dot(z, w3[...]) + b3[...]
        z = jnp.tanh(_ln(z, g3[...], e3[...]))
        z = _dot(z, w4[...]) + b4[...]
        upd = jnp.concatenate(
            [z, jnp.zeros((z.shape[0], HP - HID), jnp.float32)], axis=1)
        o_ref[...] = a8 + upd

    return body


def _full(shape):
    return pl.BlockSpec(shape, lambda i: (0,) * len(shape))


def _row_blk(rows, cols):
    return pl.BlockSpec((rows, cols), lambda i: (i, 0))


def _inp_call(x, w, b, g, be):
    return pl.pallas_call(
        _inp_body,
        grid=(N // BN,),
        in_specs=[_row_blk(BN, IN_CH), _full(w.shape), _full(b.shape),
                  _full(g.shape), _full(be.shape)],
        out_specs=_row_blk(BN, HP),
        out_shape=jax.ShapeDtypeStruct((N, HP), jnp.float32),
    )(x, w, b, g, be)


def _gx_call(xs, xe, wxs, wxt):
    return pl.pallas_call(
        _gx_body,
        grid=(EGRID,),
        in_specs=[_row_blk(BE, IN_CH), _row_blk(BE, IN_CH),
                  _full(wxs.shape), _full(wxt.shape)],
        out_specs=_row_blk(BE, D),
        out_shape=jax.ShapeDtypeStruct((E, D), jnp.float32),
    )(xs, xe, wxs, wxt)


def _edge_loop_call(ck, a8s, a8t, gx, xs, xe, ws):
    return pl.pallas_call(
        _make_edge_loop_body(ck),
        grid=(EGRID,),
        in_specs=[_row_blk(BE, HP), _row_blk(BE, HP), _row_blk(BE, D),
                  _row_blk(BE, IN_CH), _row_blk(BE, IN_CH)]
        + [_full(w.shape) for w in ws],
        out_specs=(_row_blk(BE, IN_CH), _row_blk(BE, HP),
                   _row_blk(BE, IN_CH), _row_blk(BE, HP)),
        out_shape=(jax.ShapeDtypeStruct((E, IN_CH), jnp.float32),
                   jax.ShapeDtypeStruct((E, HP), jnp.float32),
                   jax.ShapeDtypeStruct((E, IN_CH), jnp.float32),
                   jax.ShapeDtypeStruct((E, HP), jnp.float32)),
    )(a8s, a8t, gx, xs, xe, *ws)


def _edge_final_call(a8s, a8t, gx, ws):
    return pl.pallas_call(
        _edge_final_body,
        grid=(EGRID,),
        in_specs=[_row_blk(BE, HP), _row_blk(BE, HP), _row_blk(BE, D)]
        + [_full(w.shape) for w in ws],
        out_specs=_row_blk(BE, 1),
        out_shape=jax.ShapeDtypeStruct((E, 1), jnp.float32),
    )(a8s, a8t, gx, *ws)


def _node_call(ck, miw, mih, mow, moh, a8, x, ws):
    return pl.pallas_call(
        _make_node_body(ck),
        grid=(N // BN,),
        in_specs=[_row_blk(BN, IN_CH), _row_blk(BN, HP),
                  _row_blk(BN, IN_CH), _row_blk(BN, HP),
                  _row_blk(BN, HP), _row_blk(BN, IN_CH)]
        + [_full(w.shape) for w in ws],
        out_specs=_row_blk(BN, HP),
        out_shape=jax.ShapeDtypeStruct((N, HP), jnp.float32),
    )(miw, mih, mow, moh, a8, x, *ws)


def _prep_edge_weights(p):
    w1 = p["Ws"][0]
    return {
        "w1s8": w1[:HID],
        "w1xs": w1[HID:D],
        "w1t8": w1[D:D + HID],
        "w1xt": w1[D + HID:],
        "rest": [p["Ws"][1], p["Ws"][2], p["Ws"][3],
                 p["bs"][0].reshape(1, D), p["bs"][1].reshape(1, D),
                 p["bs"][2].reshape(1, D), p["bs"][3].reshape(1, 1),
                 p["gs"][0].reshape(1, D), p["gs"][1].reshape(1, D),
                 p["gs"][2].reshape(1, D),
                 p["bes"][0].reshape(1, D), p["bes"][1].reshape(1, D),
                 p["bes"][2].reshape(1, D)],
    }


def _prep_node_weights(p):
    w1 = p["Ws"][0]
    return [w1[:HID], w1[HID:D],
            w1[D:D + HID], w1[D + HID:2 * D],
            w1[2 * D:2 * D + HID], w1[2 * D + HID:],
            p["Ws"][1], p["Ws"][2], p["Ws"][3],
            p["bs"][0].reshape(1, HID), p["bs"][1].reshape(1, HID),
            p["bs"][2].reshape(1, HID), p["bs"][3].reshape(1, HID),
            p["gs"][0].reshape(1, HID), p["gs"][1].reshape(1, HID),
            p["gs"][2].reshape(1, HID),
            p["bes"][0].reshape(1, HID), p["bes"][1].reshape(1, HID),
            p["bes"][2].reshape(1, HID)]


def kernel(x, edge_index, params):
    start = edge_index[0]
    end = edge_index[1]
    s2 = start.reshape(E // CHUNK, CHUNK)
    t2 = end.reshape(E // CHUNK, CHUNK)
    zzw = jnp.zeros((CHUNK, IN_CH), jnp.float32)
    zzh = jnp.zeros((CHUNK, HP), jnp.float32)

    pi = params["inp"]
    ewp = _prep_edge_weights(params["edge"])
    ew = [ewp["w1s8"], ewp["w1t8"]] + ewp["rest"]
    nw = _prep_node_weights(params["node"])

    a8 = _inp_call(x, pi["Ws"][0], pi["bs"][0].reshape(1, HID),
                   pi["gs"][0].reshape(1, HID), pi["bes"][0].reshape(1, HID))
    xs, xe = _sc_gather_x(x, s2, t2)
    gx = _gx_call(xs, xe, ewp["w1xs"], ewp["w1xt"])

    for it in range(NITER):
        ck = float(it + 1)
        a8s, a8t = _sc_gather_a8(a8, s2, t2)
        msw, msh, mtw, mth = _edge_loop_call(ck, a8s, a8t, gx, xs, xe, ew)
        miw, mih, mow, moh = _sc_scatter(msw, msh, mtw, mth, s2, t2, zzw, zzh)
        a8 = _node_call(ck, miw, mih, mow, moh, a8, x, nw)

    a8s, a8t = _sc_gather_a8(a8, s2, t2)
    logit = _edge_final_call(a8s, a8t, gx, ew)
    return logit[:, 0]


# speedup vs baseline: 1.1351x; 1.1351x over previous
import functools

import jax
import jax.numpy as jnp
from jax import lax
from jax.experimental import pallas as pl
from jax.experimental.pallas import tpu as pltpu
from jax.experimental.pallas import tpu_sc as plsc

N = 10000
E = 320000
IN_CH = 128
HID = 8
D = IN_CH + HID
HP = 16
NITER = 3
EPS = 1e-5

CHUNK = 128
KSUB = 4
MACRO = CHUNK * KSUB
NMACRO = E // MACRO
NCHUNKS = E // CHUNK
NWORKER = 32
NTILE = 16

_mesh = plsc.VectorSubcoreMesh(core_axis_name="c", subcore_axis_name="s")
_sc_params = pltpu.CompilerParams(use_tc_tiling_on_sc=False)
_sc_params_tc = pltpu.CompilerParams(use_tc_tiling_on_sc=True)


@functools.partial(
    pl.kernel,
    out_type=(jax.ShapeDtypeStruct((E, IN_CH), jnp.float32),
              jax.ShapeDtypeStruct((E, IN_CH), jnp.float32)),
    mesh=_mesh,
    scratch_types=[
        pltpu.VMEM((KSUB, CHUNK), jnp.int32),
        pltpu.VMEM((MACRO, IN_CH), jnp.float32),
        pltpu.SemaphoreType.DMA,
    ],
    compiler_params=_sc_params_tc,
)
def _sc_gather_x(x_hbm, s2_hbm, t2_hbm, xs_hbm, xe_hbm, idx_v, rows_v, sem):
    wid = lax.axis_index("s") * 2 + lax.axis_index("c")
    niter = (NMACRO + NWORKER - 1) // NWORKER

    def body(i, carry):
        m = wid + i * NWORKER

        @pl.when(m < NMACRO)
        def _():
            for idx_hbm, out_hbm in ((s2_hbm, xs_hbm), (t2_hbm, xe_hbm)):
                pltpu.sync_copy(idx_hbm.at[pl.ds(m * KSUB, KSUB)], idx_v)
                copies = [
                    pltpu.async_copy(x_hbm.at[idx_v.at[j]],
                                     rows_v.at[pl.ds(j * CHUNK, CHUNK)], sem)
                    for j in range(KSUB)
                ]
                for cp in copies:
                    cp.wait()
                pltpu.sync_copy(rows_v, out_hbm.at[pl.ds(m * MACRO, MACRO)])

        return carry

    lax.fori_loop(0, niter, body, 0)


@functools.partial(
    pl.kernel,
    out_type=(jax.ShapeDtypeStruct((E, HP), jnp.float32),
              jax.ShapeDtypeStruct((E, HP), jnp.float32)),
    mesh=_mesh,
    scratch_types=[
        pltpu.VMEM((KSUB, CHUNK), jnp.int32),
        pltpu.VMEM((MACRO, HP), jnp.float32),
        pltpu.SemaphoreType.DMA,
    ],
    compiler_params=_sc_params,
)
def _sc_gather_a8(a_hbm, s2_hbm, t2_hbm, as_hbm, ae_hbm, idx_v, rows_v, sem):
    wid = lax.axis_index("s") * 2 + lax.axis_index("c")
    niter = (NMACRO + NWORKER - 1) // NWORKER

    def body(i, carry):
        m = wid + i * NWORKER

        @pl.when(m < NMACRO)
        def _():
            for idx_hbm, out_hbm in ((s2_hbm, as_hbm), (t2_hbm, ae_hbm)):
                pltpu.sync_copy(idx_hbm.at[pl.ds(m * KSUB, KSUB)], idx_v)
                copies = [
                    pltpu.async_copy(a_hbm.at[idx_v.at[j]],
                                     rows_v.at[pl.ds(j * CHUNK, CHUNK)], sem)
                    for j in range(KSUB)
                ]
                for cp in copies:
                    cp.wait()
                pltpu.sync_copy(rows_v, out_hbm.at[pl.ds(m * MACRO, MACRO)])

        return carry

    lax.fori_loop(0, niter, body, 0)


_ROWS_PER_TILE = N // NTILE
_WCHUNK = 125


_SNITER = (NCHUNKS + NTILE - 1) // NTILE


@functools.partial(
    pl.kernel,
    out_type=(jax.ShapeDtypeStruct((N, IN_CH), jnp.float32),
              jax.ShapeDtypeStruct((N, HP), jnp.float32),
              jax.ShapeDtypeStruct((N, IN_CH), jnp.float32),
              jax.ShapeDtypeStruct((N, HP), jnp.float32)),
    mesh=_mesh,
    scratch_types=[
        pltpu.VMEM((2, CHUNK), jnp.int32),
        pltpu.VMEM((2, CHUNK, IN_CH), jnp.float32),
        pltpu.VMEM((2, CHUNK, HP), jnp.float32),
        pltpu.VMEM_SHARED((N, IN_CH), jnp.float32),
        pltpu.VMEM_SHARED((N, HP), jnp.float32),
        pltpu.SemaphoreType.DMA,
        pltpu.SemaphoreType.DMA,
    ],
    compiler_params=_sc_params,
)
def _sc_scatter(msw_hbm, msh_hbm, mtw_hbm, mth_hbm, s2_hbm, t2_hbm,
                zzw_hbm, zzh_hbm, miw_hbm, mih_hbm, mow_hbm, moh_hbm,
                idx_v, roww_v, rowh_v, accw_sh, acch_sh, sem0, sem1):
    cid = lax.axis_index("c")
    sid = lax.axis_index("s")
    tbase = sid * _ROWS_PER_TILE
    sems = (sem0, sem1)

    pltpu.sync_copy(zzw_hbm, roww_v.at[0])
    pltpu.sync_copy(zzh_hbm, rowh_v.at[0])
    for k in range(_ROWS_PER_TILE // _WCHUNK):
        pltpu.sync_copy(roww_v.at[0, pl.ds(0, _WCHUNK)],
                        accw_sh.at[pl.ds(tbase + k * _WCHUNK, _WCHUNK)])
        pltpu.sync_copy(rowh_v.at[0, pl.ds(0, _WCHUNK)],
                        acch_sh.at[pl.ds(tbase + k * _WCHUNK, _WCHUNK)])
    plsc.subcore_barrier()

    def _start(i, b):
        m = sid + i * NTILE

        @pl.when(m < NCHUNKS)
        def _():
            @pl.when(cid == 0)
            def _():
                pltpu.async_copy(t2_hbm.at[pl.ds(m, 1)],
                                 idx_v.at[pl.ds(b, 1)], sems[b])
                pltpu.async_copy(msw_hbm.at[pl.ds(m * CHUNK, CHUNK)],
                                 roww_v.at[b], sems[b])
                pltpu.async_copy(msh_hbm.at[pl.ds(m * CHUNK, CHUNK)],
                                 rowh_v.at[b], sems[b])

            @pl.when(cid == 1)
            def _():
                pltpu.async_copy(s2_hbm.at[pl.ds(m, 1)],
                                 idx_v.at[pl.ds(b, 1)], sems[b])
                pltpu.async_copy(mtw_hbm.at[pl.ds(m * CHUNK, CHUNK)],
                                 roww_v.at[b], sems[b])
                pltpu.async_copy(mth_hbm.at[pl.ds(m * CHUNK, CHUNK)],
                                 rowh_v.at[b], sems[b])

    def _wait(i, b):
        m = sid + i * NTILE

        @pl.when(m < NCHUNKS)
        def _():
            pltpu.make_async_copy(t2_hbm.at[pl.ds(0, 1)],
                                  idx_v.at[pl.ds(b, 1)], sems[b]).wait()
            pltpu.make_async_copy(msw_hbm.at[pl.ds(0, CHUNK)],
                                  roww_v.at[b], sems[b]).wait()
            pltpu.make_async_copy(msh_hbm.at[pl.ds(0, CHUNK)],
                                  rowh_v.at[b], sems[b]).wait()

    def _use(i, b):
        m = sid + i * NTILE

        @pl.when(m < NCHUNKS)
        def _():
            pltpu.sync_copy(roww_v.at[b], accw_sh.at[idx_v.at[b]], add=True)
            pltpu.sync_copy(rowh_v.at[b], acch_sh.at[idx_v.at[b]], add=True)

    _start(0, 0)
    _start(1, 1)

    def body(j, carry):
        for b in (0, 1):
            i = 2 * j + b
            _wait(i, b)
            _use(i, b)
            _start(i + 2, b)
        return carry

    lax.fori_loop(0, (_SNITER + 1) // 2, body, 0)
    plsc.subcore_barrier()

    for k in range(_ROWS_PER_TILE // _WCHUNK):
        pltpu.sync_copy(accw_sh.at[pl.ds(tbase + k * _WCHUNK, _WCHUNK)],
                        roww_v.at[0, pl.ds(0, _WCHUNK)])
        pltpu.sync_copy(acch_sh.at[pl.ds(tbase + k * _WCHUNK, _WCHUNK)],
                        rowh_v.at[0, pl.ds(0, _WCHUNK)])

        @pl.when(cid == 0)
        def _():
            pltpu.sync_copy(roww_v.at[0, pl.ds(0, _WCHUNK)],
                            miw_hbm.at[pl.ds(tbase + k * _WCHUNK, _WCHUNK)])
            pltpu.sync_copy(rowh_v.at[0, pl.ds(0, _WCHUNK)],
                            mih_hbm.at[pl.ds(tbase + k * _WCHUNK, _WCHUNK)])

        @pl.when(cid == 1)
        def _():
            pltpu.sync_copy(roww_v.at[0, pl.ds(0, _WCHUNK)],
                            mow_hbm.at[pl.ds(tbase + k * _WCHUNK, _WCHUNK)])
            pltpu.sync_copy(rowh_v.at[0, pl.ds(0, _WCHUNK)],
                            moh_hbm.at[pl.ds(tbase + k * _WCHUNK, _WCHUNK)])


BN = 1000
BE = 4000
EGRID = E // BE


def _ln(z, g, b):
    m = jnp.mean(z, axis=-1, keepdims=True)
    v = jnp.var(z, axis=-1, keepdims=True)
    return (z - m) * lax.rsqrt(v + EPS) * g + b


def _dot(a, b):
    return jnp.dot(a, b, preferred_element_type=jnp.float32)


def _inp_body(x_ref, w_ref, b_ref, g_ref, be_ref, o_ref):
    xb = x_ref[...]
    z = _dot(xb, w_ref[...]) + b_ref[...]
    z = jnp.tanh(_ln(z, g_ref[...], be_ref[...]))
    o_ref[...] = jnp.concatenate(
        [z, jnp.zeros((xb.shape[0], HP - HID), jnp.float32)], axis=1)


def _gx_body(xs_ref, xe_ref, wxs_ref, wxt_ref, o_ref):
    o_ref[...] = (_dot(xs_ref[...], wxs_ref[...])
                  + _dot(xe_ref[...], wxt_ref[...]))


def _edge_mlp(a8s, a8t, gx, ck, w1s8, w1t8, w2, w3, w4, b1, b2, b3, b4,
              g1, g2, g3, e1, e2, e3):
    z = (_dot(a8s[:, :HID], w1s8) + _dot(a8t[:, :HID], w1t8)
         + ck * gx + b1)
    z = jnp.tanh(_ln(z, g1, e1))
    z = _dot(z, w2) + b2
    z = jnp.tanh(_ln(z, g2, e2))
    z = _dot(z, w3) + b3
    z = jnp.tanh(_ln(z, g3, e3))
    return _dot(z, w4) + b4


def _make_edge_loop_body(ck):
    def body(a8s_ref, a8t_ref, gx_ref, xs_ref, xe_ref,
             w1s8, w1t8, w2, w3, w4, b1, b2, b3, b4, g1, g2, g3, e1, e2, e3,
             msw_ref, msh_ref, mtw_ref, mth_ref):
        a8s = a8s_ref[...]
        a8t = a8t_ref[...]
        logit = _edge_mlp(a8s, a8t, gx_ref[...], ck,
                          w1s8[...], w1t8[...], w2[...], w3[...], w4[...],
                          b1[...], b2[...], b3[...], b4[...], g1[...], g2[...],
                          g3[...], e1[...], e2[...], e3[...])
        e = jax.nn.sigmoid(logit)
        msw_ref[...] = (e * ck) * xs_ref[...]
        msh_ref[...] = e * a8s
        mtw_ref[...] = (e * ck) * xe_ref[...]
        mth_ref[...] = e * a8t

    return body


def _edge_final_body(a8s_ref, a8t_ref, gx_ref,
                     w1s8, w1t8, w2, w3, w4, b1, b2, b3, b4,
                     g1, g2, g3, e1, e2, e3, o_ref):
    ck = float(NITER + 1)
    o_ref[...] = _edge_mlp(a8s_ref[...], a8t_ref[...], gx_ref[...], ck,
                           w1s8[...], w1t8[...], w2[...], w3[...], w4[...],
                           b1[...], b2[...], b3[...], b4[...], g1[...],
                           g2[...], g3[...], e1[...], e2[...], e3[...])


def _make_node_body(ck):
    def body(miw_ref, mih_ref, mow_ref, moh_ref, a8_ref, x_ref,
             w1a8, w1aw, w1b8, w1bw, w1c8, w1cw, w2, w3, w4,
             b1, b2, b3, b4, g1, g2, g3, e1, e2, e3, o_ref):
        a8 = a8_ref[...]
        z = (_dot(mih_ref[...][:, :HID], w1a8[...])
             + _dot(miw_ref[...], w1aw[...])
             + _dot(moh_ref[...][:, :HID], w1b8[...])
             + _dot(mow_ref[...], w1bw[...])
             + _dot(a8[:, :HID], w1c8[...])
             + ck * _dot(x_ref[...], w1cw[...])
             + b1[...])
        z = jnp.tanh(_ln(z, g1[...], e1[...]))
        z = _dot(z, w2[...]) + b2[...]
        z = jnp.tanh(_ln(z, g2[...], e2[...]))
        z = _dot(z, w3[...]) + b3[...]
        z = jnp.tanh(_ln(z, g3[...], e3[...]))
        z = _dot(z, w4[...]) + b4[...]
        upd = jnp.concatenate(
            [z, jnp.zeros((z.shape[0], HP - HID), jnp.float32)], axis=1)
        o_ref[...] = a8 + upd

    return body


def _full(shape):
    return pl.BlockSpec(shape, lambda i: (0,) * len(shape))


def _row_blk(rows, cols):
    return pl.BlockSpec((rows, cols), lambda i: (i, 0))


def _inp_call(x, w, b, g, be):
    return pl.pallas_call(
        _inp_body,
        grid=(N // BN,),
        in_specs=[_row_blk(BN, IN_CH), _full(w.shape), _full(b.shape),
                  _full(g.shape), _full(be.shape)],
        out_specs=_row_blk(BN, HP),
        out_shape=jax.ShapeDtypeStruct((N, HP), jnp.float32),
    )(x, w, b, g, be)


def _gx_call(xs, xe, wxs, wxt):
    return pl.pallas_call(
        _gx_body,
        grid=(EGRID,),
        in_specs=[_row_blk(BE, IN_CH), _row_blk(BE, IN_CH),
                  _full(wxs.shape), _full(wxt.shape)],
        out_specs=_row_blk(BE, D),
        out_shape=jax.ShapeDtypeStruct((E, D), jnp.float32),
    )(xs, xe, wxs, wxt)


def _edge_loop_call(ck, a8s, a8t, gx, xs, xe, ws):
    return pl.pallas_call(
        _make_edge_loop_body(ck),
        grid=(EGRID,),
        in_specs=[_row_blk(BE, HP), _row_blk(BE, HP), _row_blk(BE, D),
                  _row_blk(BE, IN_CH), _row_blk(BE, IN_CH)]
        + [_full(w.shape) for w in ws],
        out_specs=(_row_blk(BE, IN_CH), _row_blk(BE, HP),
                   _row_blk(BE, IN_CH), _row_blk(BE, HP)),
        out_shape=(jax.ShapeDtypeStruct((E, IN_CH), jnp.float32),
                   jax.ShapeDtypeStruct((E, HP), jnp.float32),
                   jax.ShapeDtypeStruct((E, IN_CH), jnp.float32),
                   jax.ShapeDtypeStruct((E, HP), jnp.float32)),
    )(a8s, a8t, gx, xs, xe, *ws)


def _edge_final_call(a8s, a8t, gx, ws):
    return pl.pallas_call(
        _edge_final_body,
        grid=(EGRID,),
        in_specs=[_row_blk(BE, HP), _row_blk(BE, HP), _row_blk(BE, D)]
        + [_full(w.shape) for w in ws],
        out_specs=_row_blk(BE, 1),
        out_shape=jax.ShapeDtypeStruct((E, 1), jnp.float32),
    )(a8s, a8t, gx, *ws)


def _node_call(ck, miw, mih, mow, moh, a8, x, ws):
    return pl.pallas_call(
        _make_node_body(ck),
        grid=(N // BN,),
        in_specs=[_row_blk(BN, IN_CH), _row_blk(BN, HP),
                  _row_blk(BN, IN_CH), _row_blk(BN, HP),
                  _row_blk(BN, HP), _row_blk(BN, IN_CH)]
        + [_full(w.shape) for w in ws],
        out_specs=_row_blk(BN, HP),
        out_shape=jax.ShapeDtypeStruct((N, HP), jnp.float32),
    )(miw, mih, mow, moh, a8, x, *ws)


def _prep_edge_weights(p):
    w1 = p["Ws"][0]
    return {
        "w1s8": w1[:HID],
        "w1xs": w1[HID:D],
        "w1t8": w1[D:D + HID],
        "w1xt": w1[D + HID:],
        "rest": [p["Ws"][1], p["Ws"][2], p["Ws"][3],
                 p["bs"][0].reshape(1, D), p["bs"][1].reshape(1, D),
                 p["bs"][2].reshape(1, D), p["bs"][3].reshape(1, 1),
                 p["gs"][0].reshape(1, D), p["gs"][1].reshape(1, D),
                 p["gs"][2].reshape(1, D),
                 p["bes"][0].reshape(1, D), p["bes"][1].reshape(1, D),
                 p["bes"][2].reshape(1, D)],
    }


def _prep_node_weights(p):
    w1 = p["Ws"][0]
    return [w1[:HID], w1[HID:D],
            w1[D:D + HID], w1[D + HID:2 * D],
            w1[2 * D:2 * D + HID], w1[2 * D + HID:],
            p["Ws"][1], p["Ws"][2], p["Ws"][3],
            p["bs"][0].reshape(1, HID), p["bs"][1].reshape(1, HID),
            p["bs"][2].reshape(1, HID), p["bs"][3].reshape(1, HID),
            p["gs"][0].reshape(1, HID), p["gs"][1].reshape(1, HID),
            p["gs"][2].reshape(1, HID),
            p["bes"][0].reshape(1, HID), p["bes"][1].reshape(1, HID),
            p["bes"][2].reshape(1, HID)]


def kernel(x, edge_index, params):
    start = edge_index[0]
    end = edge_index[1]
    s2 = start.reshape(E // CHUNK, CHUNK)
    t2 = end.reshape(E // CHUNK, CHUNK)
    zzw = jnp.zeros((CHUNK, IN_CH), jnp.float32)
    zzh = jnp.zeros((CHUNK, HP), jnp.float32)

    pi = params["inp"]
    ewp = _prep_edge_weights(params["edge"])
    ew = [ewp["w1s8"], ewp["w1t8"]] + ewp["rest"]
    nw = _prep_node_weights(params["node"])

    a8 = _inp_call(x, pi["Ws"][0], pi["bs"][0].reshape(1, HID),
                   pi["gs"][0].reshape(1, HID), pi["bes"][0].reshape(1, HID))
    xs, xe = _sc_gather_x(x, s2, t2)
    gx = _gx_call(xs, xe, ewp["w1xs"], ewp["w1xt"])

    for it in range(NITER):
        ck = float(it + 1)
        a8s, a8t = _sc_gather_a8(a8, s2, t2)
        msw, msh, mtw, mth = _edge_loop_call(ck, a8s, a8t, gx, xs, xe, ew)
        miw, mih, mow, moh = _sc_scatter(msw, msh, mtw, mth, s2, t2, zzw, zzh)
        a8 = _node_call(ck, miw, mih, mow, moh, a8, x, nw)

    a8s, a8t = _sc_gather_a8(a8, s2, t2)
    logit = _edge_final_call(a8s, a8t, gx, ew)
    return logit[:, 0]
